# initial kernel scaffold (unmeasured)
import jax
import jax.numpy as jnp
from jax import lax
from jax.experimental import pallas as pl
from jax.experimental.pallas import tpu as pltpu

T = 4096
D = 2048
V_LOCAL = 8192
N_CHUNKS = 8
ROWS = T // N_CHUNKS
N_SLOTS = 4


def _exchange_body(x_ref, o_ref, comm_ref, send_sems, recv_sems, credit_sem):
    my_x = lax.axis_index("x")
    my_y = lax.axis_index("y")
    my_z = lax.axis_index("z")
    partner = (1 - my_x, my_y, my_z)

    barrier = pltpu.get_barrier_semaphore()
    pl.semaphore_signal(
        barrier, inc=1, device_id=partner, device_id_type=pl.DeviceIdType.MESH
    )
    pl.semaphore_wait(barrier, 1)

    for k in range(N_CHUNKS):
        slot = k % N_SLOTS
        if k >= N_SLOTS:
            pl.semaphore_wait(credit_sem, 1)
        rdma = pltpu.make_async_remote_copy(
            src_ref=x_ref.at[pl.ds(k * ROWS, ROWS), :],
            dst_ref=comm_ref.at[slot],
            send_sem=send_sems.at[slot],
            recv_sem=recv_sems.at[slot],
            device_id=partner,
            device_id_type=pl.DeviceIdType.MESH,
        )
        rdma.start()
        rdma.wait_recv()
        rdma.wait_send()
        o_ref[pl.ds(k * ROWS, ROWS), :] = (
            x_ref[pl.ds(k * ROWS, ROWS), :] + comm_ref[slot]
        )
        if k < N_SLOTS:
            pl.semaphore_signal(
                credit_sem,
                inc=1,
                device_id=partner,
                device_id_type=pl.DeviceIdType.MESH,
            )


def kernel(ids, E):
    my_x = lax.axis_index("x")
    local = ids - my_x * V_LOCAL
    valid = (local >= 0) & (local < V_LOCAL)
    partial = jnp.where(
        valid[:, None], E[jnp.clip(local, 0, V_LOCAL - 1)], jnp.float32(0)
    )

    return pl.pallas_call(
        _exchange_body,
        out_shape=jax.ShapeDtypeStruct((T, D), jnp.float32),
        in_specs=[pl.BlockSpec(memory_space=pltpu.VMEM)],
        out_specs=pl.BlockSpec(memory_space=pltpu.VMEM),
        scratch_shapes=[
            pltpu.VMEM((N_SLOTS, ROWS, D), jnp.float32),
            pltpu.SemaphoreType.DMA((N_SLOTS,)),
            pltpu.SemaphoreType.DMA((N_SLOTS,)),
            pltpu.SemaphoreType.REGULAR,
        ],
        input_output_aliases={0: 0},
        compiler_params=pltpu.CompilerParams(collective_id=0),
    )(partial)


# baseline (device time: 3885354 ns/iter reference)
import jax
import jax.numpy as jnp
from jax import lax
from jax.experimental import pallas as pl
from jax.experimental.pallas import tpu as pltpu

T = 4096
D = 2048
V_LOCAL = 8192
N_CHUNKS = 8
ROWS = T // N_CHUNKS
N_SLOTS = 4


def _exchange_body(
    x_ref, o_ref, comm_ref, stage_ref, send_sems, recv_sems, copy_sems, credit_sem
):
    my_x = lax.axis_index("x")
    my_y = lax.axis_index("y")
    my_z = lax.axis_index("z")
    partner = (1 - my_x, my_y, my_z)

    barrier = pltpu.get_barrier_semaphore()
    pl.semaphore_signal(
        barrier, inc=1, device_id=partner, device_id_type=pl.DeviceIdType.MESH
    )
    pl.semaphore_wait(barrier, 1)

    out_copies = [None, None]
    for k in range(N_CHUNKS):
        slot = k % N_SLOTS
        s2 = k % 2
        if k >= N_SLOTS:
            pl.semaphore_wait(credit_sem, 1)
        rdma = pltpu.make_async_remote_copy(
            src_ref=x_ref.at[pl.ds(k * ROWS, ROWS), :],
            dst_ref=comm_ref.at[slot],
            send_sem=send_sems.at[slot],
            recv_sem=recv_sems.at[slot],
            device_id=partner,
            device_id_type=pl.DeviceIdType.MESH,
        )
        rdma.start()
        rdma.wait_recv()
        if out_copies[s2] is not None:
            out_copies[s2].wait()
        stage_ref[s2] = x_ref[pl.ds(k * ROWS, ROWS), :] + comm_ref[slot]
        cp = pltpu.make_async_copy(
            stage_ref.at[s2],
            o_ref.at[pl.ds(k * ROWS, ROWS), :],
            copy_sems.at[s2],
        )
        cp.start()
        out_copies[s2] = cp
        if k < N_SLOTS:
            pl.semaphore_signal(
                credit_sem,
                inc=1,
                device_id=partner,
                device_id_type=pl.DeviceIdType.MESH,
            )
        rdma.wait_send()

    for cp in out_copies:
        cp.wait()


def kernel(ids, E):
    my_x = lax.axis_index("x")
    local = ids - my_x * V_LOCAL
    valid = (local >= 0) & (local < V_LOCAL)
    partial = jnp.where(
        valid[:, None], E[jnp.clip(local, 0, V_LOCAL - 1)], jnp.float32(0)
    )

    return pl.pallas_call(
        _exchange_body,
        out_shape=jax.ShapeDtypeStruct((T, D), jnp.float32),
        in_specs=[pl.BlockSpec(memory_space=pltpu.VMEM)],
        out_specs=pl.BlockSpec(memory_space=pl.ANY),
        scratch_shapes=[
            pltpu.VMEM((N_SLOTS, ROWS, D), jnp.float32),
            pltpu.VMEM((2, ROWS, D), jnp.float32),
            pltpu.SemaphoreType.DMA((N_SLOTS,)),
            pltpu.SemaphoreType.DMA((N_SLOTS,)),
            pltpu.SemaphoreType.DMA((2,)),
            pltpu.SemaphoreType.REGULAR,
        ],
        compiler_params=pltpu.CompilerParams(
            collective_id=0, vmem_limit_bytes=60 * 1024 * 1024
        ),
    )(partial)


# device time: 453827 ns/iter; 8.5613x vs baseline; 8.5613x over previous
import jax
import jax.numpy as jnp
from jax import lax
from jax.experimental import pallas as pl
from jax.experimental.pallas import tpu as pltpu

T = 4096
D = 2048
V_LOCAL = 8192
N_CHUNKS = 8
ROWS = T // N_CHUNKS
N_SLOTS = 4


def _body(
    ids_ref,
    mask_ref,
    e_ref,
    o_ref,
    gbuf,
    comm_ref,
    stage_ref,
    gsems,
    send_sems,
    recv_sems,
    copy_sems,
    credit_sem,
):
    my_x = lax.axis_index("x")
    my_y = lax.axis_index("y")
    my_z = lax.axis_index("z")
    partner = (1 - my_x, my_y, my_z)
    off = my_x * V_LOCAL

    barrier = pltpu.get_barrier_semaphore()
    pl.semaphore_signal(
        barrier, inc=1, device_id=partner, device_id_type=pl.DeviceIdType.MESH
    )
    pl.semaphore_wait(barrier, 1)

    def issue_gather(k):
        gs = k % 2

        def row(r, _):
            lidx = ids_ref[k * ROWS + r] - off
            lidx = jnp.clip(lidx, 0, V_LOCAL - 1)
            pltpu.make_async_copy(
                e_ref.at[pl.ds(lidx, 1), :],
                gbuf.at[gs, pl.ds(r, 1), :],
                gsems.at[gs],
            ).start()
            return 0

        lax.fori_loop(0, ROWS, row, 0)

    def wait_gather(k):
        gs = k % 2

        def row(r, _):
            pltpu.make_async_copy(
                e_ref.at[pl.ds(0, 1), :],
                gbuf.at[gs, pl.ds(0, 1), :],
                gsems.at[gs],
            ).wait()
            return 0

        lax.fori_loop(0, ROWS, row, 0)

    def make_rdma(k):
        slot = k % N_SLOTS
        return pltpu.make_async_remote_copy(
            src_ref=gbuf.at[k % 2],
            dst_ref=comm_ref.at[slot],
            send_sem=send_sems.at[slot],
            recv_sem=recv_sems.at[slot],
            device_id=partner,
            device_id_type=pl.DeviceIdType.MESH,
        )

    rdmas = [make_rdma(k) for k in range(N_CHUNKS)]
    out_copies = [None, None]

    issue_gather(0)
    for k in range(N_CHUNKS):
        slot = k % N_SLOTS
        s2 = k % 2
        wait_gather(k)
        if k >= N_SLOTS:
            pl.semaphore_wait(credit_sem, 1)
        rdmas[k].start()
        if k + 1 < N_CHUNKS:
            if k - 1 >= 0:
                rdmas[k - 1].wait_send()
            issue_gather(k + 1)
        rdmas[k].wait_recv()
        if out_copies[s2] is not None:
            out_copies[s2].wait()
        m = mask_ref[pl.ds(k * ROWS, ROWS), :]
        stage_ref[s2] = gbuf[s2] * m + comm_ref[slot] * (1.0 - m)
        cp = pltpu.make_async_copy(
            stage_ref.at[s2],
            o_ref.at[pl.ds(k * ROWS, ROWS), :],
            copy_sems.at[s2],
        )
        cp.start()
        out_copies[s2] = cp
        if k < N_SLOTS:
            pl.semaphore_signal(
                credit_sem,
                inc=1,
                device_id=partner,
                device_id_type=pl.DeviceIdType.MESH,
            )

    rdmas[N_CHUNKS - 2].wait_send()
    rdmas[N_CHUNKS - 1].wait_send()
    for cp in out_copies:
        cp.wait()


def kernel(ids, E):
    my_x = lax.axis_index("x")
    local = ids - my_x * V_LOCAL
    mask = ((local >= 0) & (local < V_LOCAL)).astype(jnp.float32)[:, None]

    return pl.pallas_call(
        _body,
        out_shape=jax.ShapeDtypeStruct((T, D), jnp.float32),
        in_specs=[
            pl.BlockSpec(memory_space=pltpu.MemorySpace.SMEM),
            pl.BlockSpec(memory_space=pltpu.MemorySpace.VMEM),
            pl.BlockSpec(memory_space=pl.ANY),
        ],
        out_specs=pl.BlockSpec(memory_space=pl.ANY),
        scratch_shapes=[
            pltpu.VMEM((2, ROWS, D), jnp.float32),
            pltpu.VMEM((N_SLOTS, ROWS, D), jnp.float32),
            pltpu.VMEM((2, ROWS, D), jnp.float32),
            pltpu.SemaphoreType.DMA((2,)),
            pltpu.SemaphoreType.DMA((N_SLOTS,)),
            pltpu.SemaphoreType.DMA((N_SLOTS,)),
            pltpu.SemaphoreType.DMA((2,)),
            pltpu.SemaphoreType.REGULAR,
        ],
        compiler_params=pltpu.CompilerParams(
            collective_id=0, vmem_limit_bytes=60 * 1024 * 1024
        ),
    )(ids, mask, E)


# device time: 273076 ns/iter; 14.2281x vs baseline; 1.6619x over previous
import jax
import jax.numpy as jnp
from jax import lax
from jax.experimental import pallas as pl
from jax.experimental.pallas import tpu as pltpu

T = 4096
D = 2048
V_LOCAL = 8192
N_CHUNKS = 8
ROWS = T // 2 // N_CHUNKS


def _body(
    ids_ref,
    mask_ref,
    e_ref,
    o_ref,
    gbuf,
    xcomm,
    zbuf,
    zrecv,
    gsems,
    xsend, xrecvs,
    zsend, zrecvs,
    ocp_sems,
    zcp_sem,
):
    my_x = lax.axis_index("x")
    my_y = lax.axis_index("y")
    my_z = lax.axis_index("z")
    xpartner = (1 - my_x, my_y, my_z)
    zpartner = (my_x, my_y, 1 - my_z)
    off = my_x * V_LOCAL
    base = my_z * (T // 2)
    pbase = (1 - my_z) * (T // 2)

    barrier = pltpu.get_barrier_semaphore()
    for nbr in (xpartner, zpartner):
        pl.semaphore_signal(
            barrier, inc=1, device_id=nbr, device_id_type=pl.DeviceIdType.MESH
        )
    pl.semaphore_wait(barrier, 2)

    def issue_gather(j):
        gs = j % 2
        row0 = base + j * ROWS

        def row(r, _):
            lidx = jnp.clip(ids_ref[row0 + r] - off, 0, V_LOCAL - 1)
            pltpu.make_async_copy(
                e_ref.at[pl.ds(lidx, 1), :],
                gbuf.at[gs, pl.ds(r, 1), :],
                gsems.at[gs],
            ).start()
            return 0

        lax.fori_loop(0, ROWS, row, 0, unroll=8)

    def wait_gather(j):
        gs = j % 2
        pltpu.make_async_copy(
            e_ref.at[pl.ds(0, ROWS), :], gbuf.at[gs], gsems.at[gs]
        ).wait()

    def make_xr(j):
        return pltpu.make_async_remote_copy(
            src_ref=gbuf.at[j % 2],
            dst_ref=xcomm.at[j],
            send_sem=xsend.at[j],
            recv_sem=xrecvs.at[j],
            device_id=xpartner,
            device_id_type=pl.DeviceIdType.MESH,
        )

    def make_zr(j):
        return pltpu.make_async_remote_copy(
            src_ref=zbuf.at[j % 2],
            dst_ref=zrecv.at[j],
            send_sem=zsend.at[j],
            recv_sem=zrecvs.at[j],
            device_id=zpartner,
            device_id_type=pl.DeviceIdType.MESH,
        )

    xr = [make_xr(j) for j in range(N_CHUNKS)]
    zr = [make_zr(j) for j in range(N_CHUNKS)]
    owncp = [None] * N_CHUNKS

    def drain_zrecv(j):
        zr[j].wait_recv()
        cp = pltpu.make_async_copy(
            zrecv.at[j],
            o_ref.at[pl.ds(pbase + j * ROWS, ROWS), :],
            zcp_sem.at[0],
        )
        cp.start()
        cp.wait()

    issue_gather(0)
    for j in range(N_CHUNKS):
        s2 = j % 2
        wait_gather(j)
        xr[j].start()
        if j + 1 < N_CHUNKS:
            if j >= 1:
                xr[j - 1].wait_send()
            issue_gather(j + 1)
        xr[j].wait_recv()
        if j >= 2:
            zr[j - 2].wait_send()
            owncp[j - 2].wait()
        m = mask_ref[pl.ds(base + j * ROWS, ROWS), :]
        zbuf[s2] = gbuf[s2] * m + xcomm[j] * (1.0 - m)
        if j >= 1:
            drain_zrecv(j - 1)
        zr[j].start()
        cp = pltpu.make_async_copy(
            zbuf.at[s2],
            o_ref.at[pl.ds(base + j * ROWS, ROWS), :],
            ocp_sems.at[s2],
        )
        cp.start()
        owncp[j] = cp

    xr[N_CHUNKS - 2].wait_send()
    xr[N_CHUNKS - 1].wait_send()
    zr[N_CHUNKS - 2].wait_send()
    zr[N_CHUNKS - 1].wait_send()
    owncp[N_CHUNKS - 2].wait()
    owncp[N_CHUNKS - 1].wait()
    drain_zrecv(N_CHUNKS - 1)


def kernel(ids, E):
    my_x = lax.axis_index("x")
    local = ids - my_x * V_LOCAL
    mask = ((local >= 0) & (local < V_LOCAL)).astype(jnp.float32)[:, None]

    return pl.pallas_call(
        _body,
        out_shape=jax.ShapeDtypeStruct((T, D), jnp.float32),
        in_specs=[
            pl.BlockSpec(memory_space=pltpu.MemorySpace.SMEM),
            pl.BlockSpec(memory_space=pltpu.MemorySpace.VMEM),
            pl.BlockSpec(memory_space=pl.ANY),
        ],
        out_specs=pl.BlockSpec(memory_space=pl.ANY),
        scratch_shapes=[
            pltpu.VMEM((2, ROWS, D), jnp.float32),
            pltpu.VMEM((N_CHUNKS, ROWS, D), jnp.float32),
            pltpu.VMEM((2, ROWS, D), jnp.float32),
            pltpu.VMEM((N_CHUNKS, ROWS, D), jnp.float32),
            pltpu.SemaphoreType.DMA((2,)),
            pltpu.SemaphoreType.DMA((N_CHUNKS,)),
            pltpu.SemaphoreType.DMA((N_CHUNKS,)),
            pltpu.SemaphoreType.DMA((N_CHUNKS,)),
            pltpu.SemaphoreType.DMA((N_CHUNKS,)),
            pltpu.SemaphoreType.DMA((2,)),
            pltpu.SemaphoreType.DMA((1,)),
        ],
        compiler_params=pltpu.CompilerParams(
            collective_id=0, vmem_limit_bytes=60 * 1024 * 1024
        ),
    )(ids, mask, E)


# device time: 272627 ns/iter; 14.2515x vs baseline; 1.0016x over previous
import jax
import jax.numpy as jnp
from jax import lax
from jax.experimental import pallas as pl
from jax.experimental.pallas import tpu as pltpu

T = 4096
D = 2048
V_LOCAL = 8192
N_CHUNKS = 8
ROWS = T // 2 // N_CHUNKS


def _body(
    ids_ref,
    nv_ref,
    mask_ref,
    e_ref,
    o_ref,
    gbuf,
    xcomm,
    zbuf,
    zrecv,
    gsems,
    xsend, xrecvs,
    zsend, zrecvs,
    ocp_sems,
    zcp_sem,
):
    my_x = lax.axis_index("x")
    my_y = lax.axis_index("y")
    my_z = lax.axis_index("z")
    xpartner = (1 - my_x, my_y, my_z)
    zpartner = (my_x, my_y, 1 - my_z)
    off = my_x * V_LOCAL
    base = my_z * (T // 2)
    pbase = (1 - my_z) * (T // 2)

    barrier = pltpu.get_barrier_semaphore()
    for nbr in (xpartner, zpartner):
        pl.semaphore_signal(
            barrier, inc=1, device_id=nbr, device_id_type=pl.DeviceIdType.MESH
        )
    pl.semaphore_wait(barrier, 2)

    def issue_gather(j):
        gs = j % 2
        row0 = base + j * ROWS

        def row(r, _):
            lidx = ids_ref[row0 + r] - off
            valid = (lidx >= 0) & (lidx < V_LOCAL)

            @pl.when(valid)
            def _():
                pltpu.make_async_copy(
                    e_ref.at[pl.ds(lidx, 1), :],
                    gbuf.at[gs, pl.ds(r, 1), :],
                    gsems.at[gs],
                ).start()

            return 0

        lax.fori_loop(0, ROWS, row, 0, unroll=8)

    def wait_gather(j):
        gs = j % 2
        nv = nv_ref[my_z * N_CHUNKS + j]

        def one(_, carry):
            pltpu.make_async_copy(
                e_ref.at[pl.ds(0, 1), :],
                gbuf.at[gs, pl.ds(0, 1), :],
                gsems.at[gs],
            ).wait()
            return carry

        lax.fori_loop(0, nv, one, 0)

    def make_xr(j):
        return pltpu.make_async_remote_copy(
            src_ref=gbuf.at[j % 2],
            dst_ref=xcomm.at[j],
            send_sem=xsend.at[j],
            recv_sem=xrecvs.at[j],
            device_id=xpartner,
            device_id_type=pl.DeviceIdType.MESH,
        )

    def make_zr(j):
        return pltpu.make_async_remote_copy(
            src_ref=zbuf.at[j % 2],
            dst_ref=zrecv.at[j],
            send_sem=zsend.at[j],
            recv_sem=zrecvs.at[j],
            device_id=zpartner,
            device_id_type=pl.DeviceIdType.MESH,
        )

    xr = [make_xr(j) for j in range(N_CHUNKS)]
    zr = [make_zr(j) for j in range(N_CHUNKS)]
    owncp = [None] * N_CHUNKS

    def drain_zrecv(j):
        zr[j].wait_recv()
        cp = pltpu.make_async_copy(
            zrecv.at[j],
            o_ref.at[pl.ds(pbase + j * ROWS, ROWS), :],
            zcp_sem.at[0],
        )
        cp.start()
        cp.wait()

    issue_gather(0)
    for j in range(N_CHUNKS):
        s2 = j % 2
        wait_gather(j)
        xr[j].start()
        if j + 1 < N_CHUNKS:
            if j >= 1:
                xr[j - 1].wait_send()
            issue_gather(j + 1)
        xr[j].wait_recv()
        if j >= 2:
            zr[j - 2].wait_send()
            owncp[j - 2].wait()
        m = mask_ref[pl.ds(base + j * ROWS, ROWS), :]
        zbuf[s2] = gbuf[s2] * m + xcomm[j] * (1.0 - m)
        if j >= 1:
            drain_zrecv(j - 1)
        zr[j].start()
        cp = pltpu.make_async_copy(
            zbuf.at[s2],
            o_ref.at[pl.ds(base + j * ROWS, ROWS), :],
            ocp_sems.at[s2],
        )
        cp.start()
        owncp[j] = cp

    xr[N_CHUNKS - 2].wait_send()
    xr[N_CHUNKS - 1].wait_send()
    zr[N_CHUNKS - 2].wait_send()
    zr[N_CHUNKS - 1].wait_send()
    owncp[N_CHUNKS - 2].wait()
    owncp[N_CHUNKS - 1].wait()
    drain_zrecv(N_CHUNKS - 1)


def kernel(ids, E):
    my_x = lax.axis_index("x")
    local = ids - my_x * V_LOCAL
    valid = (local >= 0) & (local < V_LOCAL)
    mask = valid.astype(jnp.float32)[:, None]
    nvalid = valid.reshape(2 * N_CHUNKS, ROWS).sum(axis=1).astype(jnp.int32)

    return pl.pallas_call(
        _body,
        out_shape=jax.ShapeDtypeStruct((T, D), jnp.float32),
        in_specs=[
            pl.BlockSpec(memory_space=pltpu.MemorySpace.SMEM),
            pl.BlockSpec(memory_space=pltpu.MemorySpace.SMEM),
            pl.BlockSpec(memory_space=pltpu.MemorySpace.VMEM),
            pl.BlockSpec(memory_space=pl.ANY),
        ],
        out_specs=pl.BlockSpec(memory_space=pl.ANY),
        scratch_shapes=[
            pltpu.VMEM((2, ROWS, D), jnp.float32),
            pltpu.VMEM((N_CHUNKS, ROWS, D), jnp.float32),
            pltpu.VMEM((2, ROWS, D), jnp.float32),
            pltpu.VMEM((N_CHUNKS, ROWS, D), jnp.float32),
            pltpu.SemaphoreType.DMA((2,)),
            pltpu.SemaphoreType.DMA((N_CHUNKS,)),
            pltpu.SemaphoreType.DMA((N_CHUNKS,)),
            pltpu.SemaphoreType.DMA((N_CHUNKS,)),
            pltpu.SemaphoreType.DMA((N_CHUNKS,)),
            pltpu.SemaphoreType.DMA((2,)),
            pltpu.SemaphoreType.DMA((1,)),
        ],
        compiler_params=pltpu.CompilerParams(
            collective_id=0, vmem_limit_bytes=60 * 1024 * 1024
        ),
    )(ids, nvalid, mask, E)


# device time: 264291 ns/iter; 14.7010x vs baseline; 1.0315x over previous
import jax
import jax.numpy as jnp
from jax import lax
from jax.experimental import pallas as pl
from jax.experimental.pallas import tpu as pltpu

T = 4096
D = 2048
V_LOCAL = 8192
N_CHUNKS = 8
ROWS = T // 2 // N_CHUNKS


def _body(
    ids_ref,
    nv_ref,
    mask_ref,
    e_ref,
    o_ref,
    gbuf,
    xcomm,
    zbuf,
    zrecv,
    gsems,
    xsend, xrecvs,
    zsend, zrecvs,
    ocp_sems,
    zcp_sem,
):
    my_x = lax.axis_index("x")
    my_y = lax.axis_index("y")
    my_z = lax.axis_index("z")
    xpartner = (1 - my_x, my_y, my_z)
    zpartner = (my_x, my_y, 1 - my_z)
    off = my_x * V_LOCAL
    base = my_z * (T // 2)
    pbase = (1 - my_z) * (T // 2)

    barrier = pltpu.get_barrier_semaphore()
    for nbr in (xpartner, zpartner):
        pl.semaphore_signal(
            barrier, inc=1, device_id=nbr, device_id_type=pl.DeviceIdType.MESH
        )
    pl.semaphore_wait(barrier, 2)

    def issue_gather(j):
        gs = j % 4
        row0 = base + j * ROWS

        def row(r, _):
            lidx = ids_ref[row0 + r] - off
            valid = (lidx >= 0) & (lidx < V_LOCAL)

            @pl.when(valid)
            def _():
                pltpu.make_async_copy(
                    e_ref.at[pl.ds(lidx, 1), :],
                    gbuf.at[gs, pl.ds(r, 1), :],
                    gsems.at[gs],
                ).start()

            return 0

        lax.fori_loop(0, ROWS, row, 0, unroll=8)

    def wait_gather(j):
        gs = j % 4
        nv = nv_ref[my_z * N_CHUNKS + j]

        def one(_, carry):
            pltpu.make_async_copy(
                e_ref.at[pl.ds(0, 1), :],
                gbuf.at[gs, pl.ds(0, 1), :],
                gsems.at[gs],
            ).wait()
            return carry

        lax.fori_loop(0, nv, one, 0)

    def make_xr(j):
        return pltpu.make_async_remote_copy(
            src_ref=gbuf.at[j % 4],
            dst_ref=xcomm.at[j],
            send_sem=xsend.at[j],
            recv_sem=xrecvs.at[j],
            device_id=xpartner,
            device_id_type=pl.DeviceIdType.MESH,
        )

    def make_zr(j):
        return pltpu.make_async_remote_copy(
            src_ref=zbuf.at[j % 2],
            dst_ref=zrecv.at[j],
            send_sem=zsend.at[j],
            recv_sem=zrecvs.at[j],
            device_id=zpartner,
            device_id_type=pl.DeviceIdType.MESH,
        )

    xr = [make_xr(j) for j in range(N_CHUNKS)]
    zr = [make_zr(j) for j in range(N_CHUNKS)]
    owncp = [None] * N_CHUNKS

    def drain_zrecv(j):
        zr[j].wait_recv()
        cp = pltpu.make_async_copy(
            zrecv.at[j],
            o_ref.at[pl.ds(pbase + j * ROWS, ROWS), :],
            zcp_sem.at[0],
        )
        cp.start()
        cp.wait()

    def process(k):
        xr[k].wait_recv()
        if k >= 2:
            zr[k - 2].wait_send()
            owncp[k - 2].wait()
        m = mask_ref[pl.ds(base + k * ROWS, ROWS), :]
        zbuf[k % 2] = gbuf[k % 4] * m + xcomm[k] * (1.0 - m)
        if k >= 1:
            drain_zrecv(k - 1)
        zr[k].start()
        cp = pltpu.make_async_copy(
            zbuf.at[k % 2],
            o_ref.at[pl.ds(base + k * ROWS, ROWS), :],
            ocp_sems.at[k % 2],
        )
        cp.start()
        owncp[k] = cp

    issue_gather(0)
    for j in range(N_CHUNKS):
        wait_gather(j)
        xr[j].start()
        if j + 1 < N_CHUNKS:
            if j >= 3:
                xr[j - 3].wait_send()
            issue_gather(j + 1)
        if j >= 1:
            process(j - 1)

    process(N_CHUNKS - 1)
    for j in range(N_CHUNKS - 4, N_CHUNKS):
        xr[j].wait_send()
    zr[N_CHUNKS - 2].wait_send()
    zr[N_CHUNKS - 1].wait_send()
    owncp[N_CHUNKS - 2].wait()
    owncp[N_CHUNKS - 1].wait()
    drain_zrecv(N_CHUNKS - 1)


def kernel(ids, E):
    my_x = lax.axis_index("x")
    local = ids - my_x * V_LOCAL
    valid = (local >= 0) & (local < V_LOCAL)
    mask = valid.astype(jnp.float32)[:, None]
    nvalid = valid.reshape(2 * N_CHUNKS, ROWS).sum(axis=1).astype(jnp.int32)

    return pl.pallas_call(
        _body,
        out_shape=jax.ShapeDtypeStruct((T, D), jnp.float32),
        in_specs=[
            pl.BlockSpec(memory_space=pltpu.MemorySpace.SMEM),
            pl.BlockSpec(memory_space=pltpu.MemorySpace.SMEM),
            pl.BlockSpec(memory_space=pltpu.MemorySpace.VMEM),
            pl.BlockSpec(memory_space=pl.ANY),
        ],
        out_specs=pl.BlockSpec(memory_space=pl.ANY),
        scratch_shapes=[
            pltpu.VMEM((4, ROWS, D), jnp.float32),
            pltpu.VMEM((N_CHUNKS, ROWS, D), jnp.float32),
            pltpu.VMEM((2, ROWS, D), jnp.float32),
            pltpu.VMEM((N_CHUNKS, ROWS, D), jnp.float32),
            pltpu.SemaphoreType.DMA((4,)),
            pltpu.SemaphoreType.DMA((N_CHUNKS,)),
            pltpu.SemaphoreType.DMA((N_CHUNKS,)),
            pltpu.SemaphoreType.DMA((N_CHUNKS,)),
            pltpu.SemaphoreType.DMA((N_CHUNKS,)),
            pltpu.SemaphoreType.DMA((2,)),
            pltpu.SemaphoreType.DMA((1,)),
        ],
        compiler_params=pltpu.CompilerParams(
            collective_id=0, vmem_limit_bytes=60 * 1024 * 1024
        ),
    )(ids, nvalid, mask, E)
